# Initial kernel scaffold; baseline (speedup 1.0000x reference)
import jax
import jax.numpy as jnp
from jax.experimental import pallas as pl


def kernel(token_ids, emb_table, pos_table):
    raise NotImplementedError("write your pallas kernel here")


# speedup vs baseline: 2.2809x; 2.2809x over previous
import functools

import jax
import jax.numpy as jnp
from jax import lax
from jax.experimental import pallas as pl
from jax.experimental.pallas import tpu as pltpu
from jax.experimental.pallas import tpu_sc as plsc

VOCAB = 100000
DIM = 64
SEQ = 200
BATCH = 4096

NUM_CORES = 2
NUM_SUBCORES = 16
NUM_WORKERS = NUM_CORES * NUM_SUBCORES
ROWS = BATCH * SEQ
ROWS_PER_WORKER = ROWS // NUM_WORKERS
CHUNK = 128
CHUNKS_PER_WORKER = ROWS_PER_WORKER // CHUNK
LANES = 16


def _build_sc_kernel():
    mesh = plsc.VectorSubcoreMesh(core_axis_name="c", subcore_axis_name="s")

    @functools.partial(
        pl.kernel,
        mesh=mesh,
        compiler_params=pltpu.CompilerParams(use_tc_tiling_on_sc=False),
        out_type=jax.ShapeDtypeStruct((ROWS, DIM), jnp.float32),
        scratch_types=[
            pltpu.VMEM((CHUNKS_PER_WORKER, CHUNK), jnp.int32),
            pltpu.VMEM((2 * SEQ, DIM), jnp.float32),
            pltpu.VMEM((CHUNK, DIM), jnp.float32),
        ],
    )
    def k(ids_hbm, emb_hbm, pos_hbm, out_hbm, idx_vm, pos_vm, buf):
        wid = lax.axis_index("s") * NUM_CORES + lax.axis_index("c")
        pltpu.sync_copy(ids_hbm.at[pl.ds(wid * CHUNKS_PER_WORKER,
                                         CHUNKS_PER_WORKER)], idx_vm)
        pltpu.sync_copy(pos_hbm.at[pl.ds(0, SEQ)], pos_vm.at[pl.ds(0, SEQ)])
        pltpu.sync_copy(pos_hbm.at[pl.ds(0, SEQ)], pos_vm.at[pl.ds(SEQ, SEQ)])

        @pl.loop(0, CHUNKS_PER_WORKER)
        def _(j):
            pltpu.sync_copy(emb_hbm.at[idx_vm.at[j]], buf)
            p0 = (j * CHUNK) % SEQ

            @pl.loop(0, CHUNK)
            def _(r):
                for c in range(DIM // LANES):
                    sl = pl.ds(c * LANES, LANES)
                    x = pos_vm.at[pl.ds(p0 + r, 1), sl][...]
                    plsc.addupdate(buf.at[pl.ds(r, 1), sl], x)

            pltpu.sync_copy(
                buf,
                out_hbm.at[pl.ds(wid * ROWS_PER_WORKER + j * CHUNK, CHUNK)])

    return k


_sc_kernel = _build_sc_kernel()


def kernel(token_ids, emb_table, pos_table):
    ids = token_ids.reshape(NUM_WORKERS * CHUNKS_PER_WORKER, CHUNK)
    ids = ids.astype(jnp.int32)
    out = _sc_kernel(ids, emb_table, pos_table)
    return out.reshape(BATCH, SEQ, DIM)


# speedup vs baseline: 2.7331x; 1.1982x over previous
import functools

import jax
import jax.numpy as jnp
from jax import lax
from jax.experimental import pallas as pl
from jax.experimental.pallas import tpu as pltpu
from jax.experimental.pallas import tpu_sc as plsc

VOCAB = 100000
DIM = 64
SEQ = 200
BATCH = 4096

NUM_CORES = 2
NUM_SUBCORES = 16
NUM_WORKERS = NUM_CORES * NUM_SUBCORES
ROWS = BATCH * SEQ
ROWS_PER_WORKER = ROWS // NUM_WORKERS
CHUNK = 128
CHUNKS_PER_WORKER = ROWS_PER_WORKER // CHUNK
LANES = 16


def _build_sc_kernel():
    mesh = plsc.VectorSubcoreMesh(core_axis_name="c", subcore_axis_name="s")

    NBUF = 4

    @functools.partial(
        pl.kernel,
        mesh=mesh,
        compiler_params=pltpu.CompilerParams(use_tc_tiling_on_sc=False),
        out_type=jax.ShapeDtypeStruct((ROWS, DIM), jnp.float32),
        scratch_types=[
            pltpu.VMEM((CHUNKS_PER_WORKER, CHUNK), jnp.int32),
            pltpu.VMEM((2 * SEQ, DIM), jnp.float32),
            [pltpu.VMEM((CHUNK, DIM), jnp.float32)] * NBUF,
            [pltpu.SemaphoreType.DMA] * NBUF,
            [pltpu.SemaphoreType.DMA] * NBUF,
        ],
    )
    def k(ids_hbm, emb_hbm, pos_hbm, out_hbm, idx_vm, pos_vm, bufs,
          gsems, wsems):
        wid = lax.axis_index("s") * NUM_CORES + lax.axis_index("c")
        out_base = wid * ROWS_PER_WORKER
        pltpu.sync_copy(ids_hbm.at[pl.ds(wid * CHUNKS_PER_WORKER,
                                         CHUNKS_PER_WORKER)], idx_vm)
        pltpu.sync_copy(pos_hbm.at[pl.ds(0, SEQ)], pos_vm.at[pl.ds(0, SEQ)])
        pltpu.sync_copy(pos_hbm.at[pl.ds(0, SEQ)], pos_vm.at[pl.ds(SEQ, SEQ)])

        def start_gather(jj, b):
            pltpu.async_copy(emb_hbm.at[idx_vm.at[jj]], bufs[b], gsems[b])

        def wait_gather(jj, b):
            pltpu.make_async_copy(emb_hbm.at[idx_vm.at[jj]], bufs[b],
                                  gsems[b]).wait()

        def out_slice(jj):
            return out_hbm.at[pl.ds(out_base + jj * CHUNK, CHUNK)]

        for b in range(NBUF):
            start_gather(b, b)

        @pl.loop(0, CHUNKS_PER_WORKER, step=NBUF)
        def _(j):
            for b in range(NBUF):
                jj = j + b
                wait_gather(jj, b)
                p0 = (jj * CHUNK) % SEQ

                @pl.loop(0, CHUNK, step=4)
                def _(r):
                    for rr in range(4):
                        for c in range(DIM // LANES):
                            sl = pl.ds(c * LANES, LANES)
                            x = pos_vm.at[pl.ds(p0 + r + rr, 1), sl][...]
                            plsc.addupdate(bufs[b].at[pl.ds(r + rr, 1), sl],
                                           x)

                pltpu.async_copy(bufs[b], out_slice(jj), wsems[b])

            for b in range(NBUF):
                jj = j + b + NBUF

                @pl.when(jj < CHUNKS_PER_WORKER)
                def _():
                    pltpu.make_async_copy(bufs[b], out_slice(jj - NBUF),
                                          wsems[b]).wait()
                    start_gather(jj, b)

        for b in range(NBUF):
            jj = CHUNKS_PER_WORKER - NBUF + b
            pltpu.make_async_copy(bufs[b], out_slice(jj), wsems[b]).wait()

    return k


_sc_kernel = _build_sc_kernel()


def kernel(token_ids, emb_table, pos_table):
    ids = token_ids.reshape(NUM_WORKERS * CHUNKS_PER_WORKER, CHUNK)
    ids = ids.astype(jnp.int32)
    out = _sc_kernel(ids, emb_table, pos_table)
    return out.reshape(BATCH, SEQ, DIM)


# speedup vs baseline: 3.6612x; 1.3396x over previous
import functools

import jax
import jax.numpy as jnp
from jax import lax
from jax.experimental import pallas as pl
from jax.experimental.pallas import tpu as pltpu
from jax.experimental.pallas import tpu_sc as plsc

VOCAB = 100000
DIM = 64
SEQ = 200
BATCH = 4096

NUM_CORES = 2
NUM_SUBCORES = 16
NUM_WORKERS = NUM_CORES * NUM_SUBCORES
ROWS = BATCH * SEQ
CHUNK = SEQ // 2
NCHUNKS = ROWS // CHUNK
CHUNKS_PER_WORKER = NCHUNKS // NUM_WORKERS
LANES = 16
NBUF = 4


def _build_sc_kernel():
    mesh = plsc.VectorSubcoreMesh(core_axis_name="c", subcore_axis_name="s")

    @functools.partial(
        pl.kernel,
        mesh=mesh,
        compiler_params=pltpu.CompilerParams(use_tc_tiling_on_sc=False),
        out_type=jax.ShapeDtypeStruct((BATCH, SEQ, DIM), jnp.float32),
        scratch_types=[
            pltpu.VMEM((CHUNKS_PER_WORKER, CHUNK), jnp.int32),
            pltpu.VMEM((SEQ, DIM), jnp.float32),
            [pltpu.VMEM((CHUNK, DIM), jnp.float32)] * NBUF,
            [pltpu.SemaphoreType.DMA] * NBUF,
            [pltpu.SemaphoreType.DMA] * NBUF,
        ],
    )
    def k(ids_hbm, emb_hbm, pos_hbm, out_hbm, idx_vm, pos_vm, bufs,
          gsems, wsems):
        wid = lax.axis_index("s") * NUM_CORES + lax.axis_index("c")
        chunk_base = wid * CHUNKS_PER_WORKER
        pltpu.sync_copy(ids_hbm.at[pl.ds(chunk_base, CHUNKS_PER_WORKER)],
                        idx_vm)
        pltpu.sync_copy(pos_hbm.at[pl.ds(0, SEQ)], pos_vm)

        def start_gather(jj, b):
            pltpu.async_copy(emb_hbm.at[idx_vm.at[jj]], bufs[b], gsems[b])

        def wait_gather(jj, b):
            pltpu.make_async_copy(emb_hbm.at[idx_vm.at[jj]], bufs[b],
                                  gsems[b]).wait()

        def out_slice(jj):
            g = chunk_base + jj
            return out_hbm.at[g // 2, pl.ds((g % 2) * CHUNK, CHUNK)]

        for b in range(NBUF):
            start_gather(b, b)

        @pl.loop(0, CHUNKS_PER_WORKER, step=NBUF)
        def _(j):
            for b in range(NBUF):
                jj = j + b
                wait_gather(jj, b)
                p0 = ((chunk_base + jj) % 2) * CHUNK

                @pl.loop(0, CHUNK, step=4)
                def _(r):
                    for rr in range(4):
                        for c in range(DIM // LANES):
                            sl = pl.ds(c * LANES, LANES)
                            x = pos_vm.at[pl.ds(p0 + r + rr, 1), sl][...]
                            plsc.addupdate(bufs[b].at[pl.ds(r + rr, 1), sl],
                                           x)

                pltpu.async_copy(bufs[b], out_slice(jj), wsems[b])

            for b in range(NBUF):
                jj = j + b + NBUF

                @pl.when(jj < CHUNKS_PER_WORKER)
                def _():
                    pltpu.make_async_copy(bufs[b], out_slice(jj - NBUF),
                                          wsems[b]).wait()
                    start_gather(jj, b)

        for b in range(NBUF):
            jj = CHUNKS_PER_WORKER - NBUF + b
            pltpu.make_async_copy(bufs[b], out_slice(jj), wsems[b]).wait()

    return k


_sc_kernel = _build_sc_kernel()


def kernel(token_ids, emb_table, pos_table):
    ids = token_ids.reshape(NCHUNKS, CHUNK).astype(jnp.int32)
    return _sc_kernel(ids, emb_table, pos_table)


# speedup vs baseline: 3.9871x; 1.0890x over previous
import functools

import jax
import jax.numpy as jnp
from jax import lax
from jax.experimental import pallas as pl
from jax.experimental.pallas import tpu as pltpu
from jax.experimental.pallas import tpu_sc as plsc

VOCAB = 100000
DIM = 64
SEQ = 200
BATCH = 4096

NUM_CORES = 2
NUM_SUBCORES = 16
NUM_WORKERS = NUM_CORES * NUM_SUBCORES
SEQS_PER_WORKER = BATCH // NUM_WORKERS
IDS_PER_WORKER = SEQS_PER_WORKER * SEQ
G0 = 128
G1 = SEQ - G0
LANES = 16
NBUF = 4


def _build_sc_kernel():
    mesh = plsc.VectorSubcoreMesh(core_axis_name="c", subcore_axis_name="s")

    @functools.partial(
        pl.kernel,
        mesh=mesh,
        compiler_params=pltpu.CompilerParams(use_tc_tiling_on_sc=False),
        out_type=jax.ShapeDtypeStruct((BATCH, SEQ, DIM), jnp.float32),
        scratch_types=[
            pltpu.VMEM((IDS_PER_WORKER,), jnp.int32),
            pltpu.VMEM((SEQ // 2, 2 * DIM), jnp.float32),
            [pltpu.VMEM((SEQ, DIM), jnp.float32)] * NBUF,
            [pltpu.SemaphoreType.DMA] * NBUF,
            [pltpu.SemaphoreType.DMA] * NBUF,
        ],
    )
    def k(ids_hbm, emb_hbm, pos_hbm, out_hbm, idx_vm, pos_vm, bufs,
          gsems, wsems):
        wid = lax.axis_index("s") * NUM_CORES + lax.axis_index("c")
        seq_base = wid * SEQS_PER_WORKER
        pltpu.sync_copy(ids_hbm.at[pl.ds(wid * IDS_PER_WORKER,
                                         IDS_PER_WORKER)], idx_vm)
        pltpu.sync_copy(pos_hbm, pos_vm)

        def start_gather(jj, b):
            base = jj * SEQ
            pltpu.async_copy(emb_hbm.at[idx_vm.at[pl.ds(base, G0)]],
                             bufs[b].at[pl.ds(0, G0)], gsems[b])
            pltpu.async_copy(emb_hbm.at[idx_vm.at[pl.ds(base + G0, G1)]],
                             bufs[b].at[pl.ds(G0, G1)], gsems[b])

        def wait_gather(jj, b):
            base = jj * SEQ
            pltpu.make_async_copy(emb_hbm.at[idx_vm.at[pl.ds(base, G0)]],
                                  bufs[b].at[pl.ds(0, G0)], gsems[b]).wait()
            pltpu.make_async_copy(emb_hbm.at[idx_vm.at[pl.ds(base + G0, G1)]],
                                  bufs[b].at[pl.ds(G0, G1)], gsems[b]).wait()

        for b in range(NBUF):
            start_gather(b, b)

        @pl.loop(0, SEQS_PER_WORKER, step=NBUF)
        def _(j):
            for b in range(NBUF):
                jj = j + b
                wait_gather(jj, b)

                @pl.loop(0, SEQ, step=4)
                def _(r):
                    r2 = r // 2
                    for rr in range(4):
                        for c in range(DIM // LANES):
                            src_col = (rr % 2) * DIM + c * LANES
                            x = pos_vm.at[pl.ds(r2 + rr // 2, 1),
                                          pl.ds(src_col, LANES)][...]
                            plsc.addupdate(
                                bufs[b].at[pl.ds(r + rr, 1),
                                           pl.ds(c * LANES, LANES)], x)

                pltpu.async_copy(bufs[b], out_hbm.at[seq_base + jj],
                                 wsems[b])

            for b in range(NBUF):
                jj = j + b + NBUF

                @pl.when(jj < SEQS_PER_WORKER)
                def _():
                    pltpu.make_async_copy(bufs[b],
                                          out_hbm.at[seq_base + jj - NBUF],
                                          wsems[b]).wait()
                    start_gather(jj, b)

        for b in range(NBUF):
            jj = SEQS_PER_WORKER - NBUF + b
            pltpu.make_async_copy(bufs[b], out_hbm.at[seq_base + jj],
                                  wsems[b]).wait()

    return k


_sc_kernel = _build_sc_kernel()


def kernel(token_ids, emb_table, pos_table):
    ids = token_ids.reshape(BATCH * SEQ).astype(jnp.int32)
    pos = pos_table[:SEQ].reshape(SEQ // 2, 2 * DIM)
    return _sc_kernel(ids, emb_table, pos)
